# baseline (speedup 1.0000x reference)
import jax
import jax.numpy as jnp
from jax import lax
from jax.experimental import pallas as pl
from jax.experimental.pallas import tpu as pltpu
from jax.experimental.pallas import tpu_sc as plsc

_D = 64
_NUM_ROWS = 1000000
_BATCH = 16384
_HIST = 50

_info = plsc.get_sparse_core_info()
_NC = _info.num_cores
_NS = _info.num_subcores
_NW = _NC * _NS
_BPW = _BATCH // _NW
_NBB = _BPW // 128
_NJ = _HIST * _NBB


def _body(idx_hbm, table_hbm, out_hbm, idx_v, idxt_v, rows0, rows1, t0, t1,
          sg0, sg1, sw0, sw1):
    wid = lax.axis_index("s") * _NC + lax.axis_index("c")
    rows = (rows0, rows1)
    tv = (t0, t1)
    sg = (sg0, sg1)
    sw = (sw0, sw1)
    iota = lax.iota(jnp.int32, 16)
    iota_h = iota * _HIST
    dtv = [(iota + 16 * q) // 8 for q in range(4)]
    dsv = [(iota + 16 * q) % 8 for q in range(4)]

    pltpu.sync_copy(idx_hbm.at[pl.ds(wid * _BPW * _HIST, _BPW * _HIST)], idx_v)

    def build(j, carry):
        h = j // _NBB
        bb = j % _NBB
        for g in range(8):
            src = iota_h + ((bb * 128 + g * 16) * _HIST + h)
            idxt_v[j, pl.ds(g * 16, 16)] = plsc.load_gather(idx_v, [src])
        return carry

    lax.fori_loop(0, _NJ, build, 0)

    def gather(j, p):
        return pltpu.make_async_copy(
            table_hbm.at[idxt_v.at[j]], rows[p], sg[p]
        )

    def write(j, p):
        h = j // _NBB
        btg = wid * _NBB + j % _NBB
        return pltpu.make_async_copy(
            tv[p].at[:, :, pl.ds(0, 128)],
            out_hbm.at[j // _NBB, :, wid * _NBB + j % _NBB, :, :], sw[p]
        )

    gather(0, 0).start()
    gather(1, 1).start()

    def step(jp, carry):
        for p in range(2):
            j = 2 * jp + p

            @pl.when(j >= 2)
            def _():
                write(j - 2, p).wait()

            gather(j, p).wait()

            def xp(i, blv):
                blvs = [blv + o for o in range(4)]
                vals = [
                    rows[p][4 * i + o, pl.ds(16 * q, 16)]
                    for o in range(4)
                    for q in range(4)
                ]
                for o in range(4):
                    for q in range(4):
                        plsc.store_scatter(
                            tv[p], [dtv[q], dsv[q], blvs[o]], vals[4 * o + q]
                        )
                return blv + 4

            lax.fori_loop(0, 32, xp, jnp.zeros((16,), jnp.int32))
            write(j, p).start()

            @pl.when(j + 2 < _NJ)
            def _():
                gather(j + 2, p).start()

        return carry

    lax.fori_loop(0, _NJ // 2, step, 0)
    write(_NJ - 2, 0).wait()
    write(_NJ - 1, 1).wait()


_NK = 7812


def _fmt_body(tt_hbm, tail_hbm, out_hbm, in0, in1, o0, o1, si0, si1, so0, so1):
    wid = lax.axis_index("s") * _NC + lax.axis_index("c")
    inv = (in0, in1)
    ov = (o0, o1)
    si = (si0, si1)
    so = (so0, so1)
    iota = lax.iota(jnp.int32, 16)
    dqs = [iota + 16 * q for q in range(4)]

    def blk(t):
        return wid + 32 * t

    def in_copy(t, p):
        k = blk(t)

        @pl.when(k < _NK)
        def _():
            pltpu.async_copy(
                tt_hbm.at[:, pl.ds(128 * k, 128)], inv[p].at[:, pl.ds(0, 128)],
                si[p],
            )

        @pl.when(k == _NK)
        def _():
            pltpu.async_copy(
                tail_hbm, inv[p].at[pl.ds(0, 32), pl.ds(0, 128)], si[p]
            )

    def in_wait(t, p):
        k = blk(t)

        @pl.when(k < _NK)
        def _():
            pltpu.make_async_copy(
                tt_hbm.at[:, pl.ds(128 * k, 128)], inv[p].at[:, pl.ds(0, 128)],
                si[p],
            ).wait()

        @pl.when(k == _NK)
        def _():
            pltpu.make_async_copy(
                tail_hbm, inv[p].at[pl.ds(0, 32), pl.ds(0, 128)], si[p]
            ).wait()

    def out_desc_full(t, p):
        return pltpu.make_async_copy(
            ov[p], out_hbm.at[pl.ds(64 * blk(t), 64), :], so[p]
        )

    def out_desc_tail(p):
        return pltpu.make_async_copy(
            inv[p].at[pl.ds(0, 32), pl.ds(0, 128)],
            out_hbm.at[pl.ds(64 * _NK, 32), :], so[p]
        )

    def out_start(t, p):
        k = blk(t)

        @pl.when(k < _NK)
        def _():
            out_desc_full(t, p).start()

        @pl.when(k == _NK)
        def _():
            out_desc_tail(p).start()

    def out_wait(t, p):
        k = blk(t)

        @pl.when(k < _NK)
        def _():
            out_desc_full(t, p).wait()

        @pl.when(k == _NK)
        def _():
            out_desc_tail(p).wait()

    def xp(nloop, p):
        def body(i, ilv):
            for o in range(4):
                il = 4 * i + o
                for q in range(4):
                    vals = plsc.load_gather(inv[p], [dqs[q], ilv + o])
                    ov[p][il // 2, pl.ds((il % 2) * 64 + 16 * q, 16)] = vals
            return ilv + 4

        lax.fori_loop(0, nloop, body, jnp.zeros((16,), jnp.int32))

    in_copy(0, 0)
    in_copy(1, 1)

    def step(tp, carry):
        for p in range(2):
            t = 2 * tp + p

            @pl.when(t >= 2)
            def _():
                out_wait(t - 2, p)

            in_wait(t, p)
            k = blk(t)

            @pl.when(k < _NK)
            def _():
                xp(32, p)

            out_start(t, p)
            in_copy(t + 2, p)
        return carry

    lax.fori_loop(0, 123, step, 0)
    out_wait(244, 0)


@jax.jit
def kernel(indices, embeddings):
    b, h = indices.shape
    idx_flat = indices.reshape(-1).astype(jnp.int32)
    mesh = plsc.VectorSubcoreMesh(core_axis_name="c", subcore_axis_name="s")
    tfmt = pl.kernel(
        _fmt_body,
        mesh=mesh,
        out_type=jax.ShapeDtypeStruct((_NUM_ROWS // 2, 2 * _D), jnp.float32),
        scratch_types=[
            pltpu.VMEM((_D, 129), jnp.float32),
            pltpu.VMEM((_D, 129), jnp.float32),
            pltpu.VMEM((_D, 128), jnp.float32),
            pltpu.VMEM((_D, 128), jnp.float32),
            pltpu.SemaphoreType.DMA,
            pltpu.SemaphoreType.DMA,
            pltpu.SemaphoreType.DMA,
            pltpu.SemaphoreType.DMA,
        ],
        compiler_params=pltpu.CompilerParams(
            use_tc_tiling_on_sc=True, needs_layout_passes=False
        ),
    )(embeddings.T, embeddings[_NK * 128:].reshape(32, 2 * _D))
    table = tfmt.reshape(_NUM_ROWS, _D)
    out5 = pl.kernel(
        _body,
        mesh=mesh,
        out_type=jax.ShapeDtypeStruct((_HIST, 8, _BATCH // 128, 8, 128),
                                      jnp.float32),
        scratch_types=[
            pltpu.VMEM((_BPW * _HIST,), jnp.int32),
            pltpu.VMEM((_NJ, 128), jnp.int32),
            pltpu.VMEM((128, _D), jnp.float32),
            pltpu.VMEM((128, _D), jnp.float32),
            pltpu.VMEM((8, 8, 129), jnp.float32),
            pltpu.VMEM((8, 8, 129), jnp.float32),
            pltpu.SemaphoreType.DMA,
            pltpu.SemaphoreType.DMA,
            pltpu.SemaphoreType.DMA,
            pltpu.SemaphoreType.DMA,
        ],
        compiler_params=pltpu.CompilerParams(
            use_tc_tiling_on_sc=False, needs_layout_passes=False
        ),
    )(idx_flat, table)
    x = jnp.transpose(out5, (0, 1, 3, 2, 4)).reshape(_HIST, _D, _BATCH)
    return jnp.transpose(x, (2, 0, 1))


# speedup vs baseline: 1.3104x; 1.3104x over previous
import jax
import jax.numpy as jnp
from jax import lax
from jax.experimental import pallas as pl
from jax.experimental.pallas import tpu as pltpu
from jax.experimental.pallas import tpu_sc as plsc

_D = 64
_NUM_ROWS = 1000000
_BATCH = 16384
_HIST = 50

_info = plsc.get_sparse_core_info()
_NC = _info.num_cores
_NS = _info.num_subcores
_NW = _NC * _NS
_BPW = _BATCH // _NW
_NBB = _BPW // 128
_NJ = _HIST * _NBB


def _body(idx_hbm, table_hbm, out_hbm, idx_v, idxt_v, rows0, rows1, t0, t1,
          sg0, sg1, sw0, sw1):
    wid = lax.axis_index("s") * _NC + lax.axis_index("c")
    rows = (rows0, rows1)
    tv = (t0, t1)
    sg = (sg0, sg1)
    sw = (sw0, sw1)
    iota = lax.iota(jnp.int32, 16)
    iota_h = iota * _HIST
    dtv = [(iota + 16 * q) // 8 for q in range(4)]
    dsv = [(iota + 16 * q) % 8 for q in range(4)]

    pltpu.sync_copy(idx_hbm.at[pl.ds(wid * _BPW * _HIST, _BPW * _HIST)], idx_v)

    def build(j, carry):
        h = j // _NBB
        bb = j % _NBB
        for g in range(8):
            src = iota_h + ((bb * 128 + g * 16) * _HIST + h)
            idxt_v[j, pl.ds(g * 16, 16)] = plsc.load_gather(idx_v, [src])
        return carry

    lax.fori_loop(0, _NJ, build, 0)

    def gather(j, p):
        return pltpu.make_async_copy(
            table_hbm.at[idxt_v.at[j]], rows[p], sg[p]
        )

    def write(j, p):
        h = j // _NBB
        btg = wid * _NBB + j % _NBB
        return pltpu.make_async_copy(
            tv[p].at[:, :, pl.ds(0, 128)],
            out_hbm.at[j // _NBB, :, wid * _NBB + j % _NBB, :, :], sw[p]
        )

    gather(0, 0).start()
    gather(1, 1).start()

    def step(jp, carry):
        for p in range(2):
            j = 2 * jp + p

            @pl.when(j >= 2)
            def _():
                write(j - 2, p).wait()

            gather(j, p).wait()

            def xp(i, blv):
                blvs = [blv + o for o in range(4)]
                vals = [
                    rows[p][4 * i + o, pl.ds(16 * q, 16)]
                    for o in range(4)
                    for q in range(4)
                ]
                for o in range(4):
                    for q in range(4):
                        plsc.store_scatter(
                            tv[p], [dtv[q], dsv[q], blvs[o]], vals[4 * o + q]
                        )
                return blv + 4

            lax.fori_loop(0, 32, xp, jnp.zeros((16,), jnp.int32))
            write(j, p).start()

            @pl.when(j + 2 < _NJ)
            def _():
                gather(j + 2, p).start()

        return carry

    lax.fori_loop(0, _NJ // 2, step, 0)
    write(_NJ - 2, 0).wait()
    write(_NJ - 1, 1).wait()


_NK = 7812


def _fmt_body(tt_hbm, tail_hbm, out_hbm, in0, in1, o0, o1, si0, si1, so0, so1):
    wid = lax.axis_index("s") * _NC + lax.axis_index("c")
    inv = (in0, in1)
    ov = (o0, o1)
    si = (si0, si1)
    so = (so0, so1)
    iota = lax.iota(jnp.int32, 16)
    dqs = [iota + 16 * q for q in range(4)]

    def blk(t):
        return wid + 32 * t

    def in_copy(t, p):
        k = blk(t)

        @pl.when(k < _NK)
        def _():
            pltpu.async_copy(
                tt_hbm.at[:, pl.ds(128 * k, 128)], inv[p].at[:, pl.ds(0, 128)],
                si[p],
            )

        @pl.when(k == _NK)
        def _():
            pltpu.async_copy(
                tail_hbm, inv[p].at[pl.ds(0, 32), pl.ds(0, 128)], si[p]
            )

    def in_wait(t, p):
        k = blk(t)

        @pl.when(k < _NK)
        def _():
            pltpu.make_async_copy(
                tt_hbm.at[:, pl.ds(128 * k, 128)], inv[p].at[:, pl.ds(0, 128)],
                si[p],
            ).wait()

        @pl.when(k == _NK)
        def _():
            pltpu.make_async_copy(
                tail_hbm, inv[p].at[pl.ds(0, 32), pl.ds(0, 128)], si[p]
            ).wait()

    def out_desc_full(t, p):
        return pltpu.make_async_copy(
            ov[p], out_hbm.at[pl.ds(64 * blk(t), 64), :], so[p]
        )

    def out_desc_tail(p):
        return pltpu.make_async_copy(
            inv[p].at[pl.ds(0, 32), pl.ds(0, 128)],
            out_hbm.at[pl.ds(64 * _NK, 32), :], so[p]
        )

    def out_start(t, p):
        k = blk(t)

        @pl.when(k < _NK)
        def _():
            out_desc_full(t, p).start()

        @pl.when(k == _NK)
        def _():
            out_desc_tail(p).start()

    def out_wait(t, p):
        k = blk(t)

        @pl.when(k < _NK)
        def _():
            out_desc_full(t, p).wait()

        @pl.when(k == _NK)
        def _():
            out_desc_tail(p).wait()

    def xp(nloop, p):
        def body(i, ilv):
            ilvs = [ilv + o for o in range(4)]
            vals = [
                plsc.load_gather(inv[p], [dqs[q], ilvs[o]])
                for o in range(4)
                for q in range(4)
            ]
            for o in range(4):
                il = 4 * i + o
                for q in range(4):
                    ov[p][il // 2, pl.ds((il % 2) * 64 + 16 * q, 16)] = (
                        vals[4 * o + q]
                    )
            return ilv + 4

        lax.fori_loop(0, nloop, body, jnp.zeros((16,), jnp.int32))

    in_copy(0, 0)
    in_copy(1, 1)

    def step(tp, carry):
        for p in range(2):
            t = 2 * tp + p

            @pl.when(t >= 2)
            def _():
                out_wait(t - 2, p)

            in_wait(t, p)
            k = blk(t)

            @pl.when(k < _NK)
            def _():
                xp(32, p)

            out_start(t, p)
            in_copy(t + 2, p)
        return carry

    lax.fori_loop(0, 123, step, 0)
    out_wait(244, 0)


@jax.jit
def kernel(indices, embeddings):
    b, h = indices.shape
    idx_flat = indices.reshape(-1).astype(jnp.int32)
    mesh = plsc.VectorSubcoreMesh(core_axis_name="c", subcore_axis_name="s")
    tfmt = pl.kernel(
        _fmt_body,
        mesh=mesh,
        out_type=jax.ShapeDtypeStruct((_NUM_ROWS // 2, 2 * _D), jnp.float32),
        scratch_types=[
            pltpu.VMEM((_D, 129), jnp.float32),
            pltpu.VMEM((_D, 129), jnp.float32),
            pltpu.VMEM((_D, 128), jnp.float32),
            pltpu.VMEM((_D, 128), jnp.float32),
            pltpu.SemaphoreType.DMA,
            pltpu.SemaphoreType.DMA,
            pltpu.SemaphoreType.DMA,
            pltpu.SemaphoreType.DMA,
        ],
        compiler_params=pltpu.CompilerParams(
            use_tc_tiling_on_sc=True, needs_layout_passes=False
        ),
    )(embeddings.T, embeddings[_NK * 128:].reshape(32, 2 * _D))
    table = tfmt.reshape(_NUM_ROWS, _D)
    out5 = pl.kernel(
        _body,
        mesh=mesh,
        out_type=jax.ShapeDtypeStruct((_HIST, 8, _BATCH // 128, 8, 128),
                                      jnp.float32),
        scratch_types=[
            pltpu.VMEM((_BPW * _HIST,), jnp.int32),
            pltpu.VMEM((_NJ, 128), jnp.int32),
            pltpu.VMEM((128, _D), jnp.float32),
            pltpu.VMEM((128, _D), jnp.float32),
            pltpu.VMEM((8, 8, 129), jnp.float32),
            pltpu.VMEM((8, 8, 129), jnp.float32),
            pltpu.SemaphoreType.DMA,
            pltpu.SemaphoreType.DMA,
            pltpu.SemaphoreType.DMA,
            pltpu.SemaphoreType.DMA,
        ],
        compiler_params=pltpu.CompilerParams(
            use_tc_tiling_on_sc=False, needs_layout_passes=False
        ),
    )(idx_flat, table)
    x = jnp.transpose(out5, (0, 1, 3, 2, 4)).reshape(_HIST, _D, _BATCH)
    return jnp.transpose(x, (2, 0, 1))


# speedup vs baseline: 2.0249x; 1.5453x over previous
import jax
import jax.numpy as jnp
from jax import lax
from jax.experimental import pallas as pl
from jax.experimental.pallas import tpu as pltpu
from jax.experimental.pallas import tpu_sc as plsc

_D = 64
_NUM_ROWS = 1000000
_BATCH = 16384
_HIST = 50

_info = plsc.get_sparse_core_info()
_NC = _info.num_cores
_NS = _info.num_subcores
_NW = _NC * _NS
_BPW = _BATCH // _NW
_NBB = _BPW // 128
_NJ = _HIST * _NBB


def _body(idx_hbm, table_hbm, out_hbm, idx_v, idxt_v, rows0, rows1, t0, t1,
          sg0, sg1, sw0, sw1):
    wid = lax.axis_index("s") * _NC + lax.axis_index("c")
    rows = (rows0, rows1)
    tv = (t0, t1)
    sg = (sg0, sg1)
    sw = (sw0, sw1)
    iota = lax.iota(jnp.int32, 16)
    iota_h = iota * _HIST
    dtv = [(iota + 16 * q) // 8 for q in range(4)]
    dsv = [(iota + 16 * q) % 8 for q in range(4)]

    pltpu.sync_copy(idx_hbm.at[pl.ds(wid * _BPW * _HIST, _BPW * _HIST)], idx_v)

    def build(j, carry):
        h = j // _NBB
        bb = j % _NBB
        for g in range(8):
            src = iota_h + ((bb * 128 + g * 16) * _HIST + h)
            idxt_v[j, pl.ds(g * 16, 16)] = plsc.load_gather(idx_v, [src])
        return carry

    lax.fori_loop(0, _NJ, build, 0)

    def gather(j, p):
        return pltpu.make_async_copy(
            table_hbm.at[idxt_v.at[j]], rows[p], sg[p]
        )

    def write(j, p):
        h = j // _NBB
        btg = wid * _NBB + j % _NBB
        return pltpu.make_async_copy(
            tv[p].at[:, :, pl.ds(0, 128)],
            out_hbm.at[j // _NBB, :, wid * _NBB + j % _NBB, :, :], sw[p]
        )

    gather(0, 0).start()
    gather(1, 1).start()

    def step(jp, carry):
        for p in range(2):
            j = 2 * jp + p

            @pl.when(j >= 2)
            def _():
                write(j - 2, p).wait()

            gather(j, p).wait()

            def xp(i, blv):
                blvs = [blv + o for o in range(4)]
                vals = [
                    rows[p][4 * i + o, pl.ds(16 * q, 16)]
                    for o in range(4)
                    for q in range(4)
                ]
                for o in range(4):
                    for q in range(4):
                        plsc.store_scatter(
                            tv[p], [dtv[q], dsv[q], blvs[o]], vals[4 * o + q]
                        )
                return blv + 4

            lax.fori_loop(0, 32, xp, jnp.zeros((16,), jnp.int32))
            write(j, p).start()

            @pl.when(j + 2 < _NJ)
            def _():
                gather(j + 2, p).start()

        return carry

    lax.fori_loop(0, _NJ // 2, step, 0)
    write(_NJ - 2, 0).wait()
    write(_NJ - 1, 1).wait()


@jax.jit
def kernel(indices, embeddings):
    b, h = indices.shape
    idx_flat = indices.reshape(-1).astype(jnp.int32)
    table = jax.lax.optimization_barrier(
        embeddings.reshape(_NUM_ROWS // 2, 2 * _D)
    ).reshape(_NUM_ROWS, _D)
    mesh = plsc.VectorSubcoreMesh(core_axis_name="c", subcore_axis_name="s")
    out5 = pl.kernel(
        _body,
        mesh=mesh,
        out_type=jax.ShapeDtypeStruct((_HIST, 8, _BATCH // 128, 8, 128),
                                      jnp.float32),
        scratch_types=[
            pltpu.VMEM((_BPW * _HIST,), jnp.int32),
            pltpu.VMEM((_NJ, 128), jnp.int32),
            pltpu.VMEM((128, _D), jnp.float32),
            pltpu.VMEM((128, _D), jnp.float32),
            pltpu.VMEM((8, 8, 129), jnp.float32),
            pltpu.VMEM((8, 8, 129), jnp.float32),
            pltpu.SemaphoreType.DMA,
            pltpu.SemaphoreType.DMA,
            pltpu.SemaphoreType.DMA,
            pltpu.SemaphoreType.DMA,
        ],
        compiler_params=pltpu.CompilerParams(
            use_tc_tiling_on_sc=False, needs_layout_passes=False
        ),
    )(idx_flat, table)
    x = jnp.transpose(out5, (0, 1, 3, 2, 4)).reshape(_HIST, _D, _BATCH)
    return jnp.transpose(x, (2, 0, 1))
